# baseline (speedup 1.0000x reference)
import functools

import jax
import jax.numpy as jnp
from jax import lax
from jax.experimental import pallas as pl
from jax.experimental.pallas import tpu as pltpu
from jax.experimental.pallas import tpu_sc as plsc

F32 = jnp.float32


def _fused_body(m_ref, x_ref, w_ref, c_ref, preq_ref, logits_ref, idx_ref,
                acc_ref, den_ref, qT_ref, sqp_ref, bestv_ref, besti_ref,
                *, NP, NJ, NK, Bb, S):
    i = pl.program_id(0)

    @pl.when(i == 0)
    def _init():
        acc_ref[...] = jnp.zeros_like(acc_ref)
        den_ref[...] = jnp.zeros_like(den_ref)
        qT_ref[...] = jnp.zeros_like(qT_ref)

    @pl.when(i < NP)
    def _pool():
        m = m_ref[...]
        x = x_ref[...]
        acc_ref[...] += jnp.sum(x * m, axis=1)
        den_ref[...] += jnp.sum(m[:, :, 0], axis=1, keepdims=True)

    @pl.when(i == NP - 1)
    def _fin_pool():
        acc_ref[...] = acc_ref[...] / jnp.clip(den_ref[...], 1.0, None)

    @pl.when((i >= NP) & (i < NP + NJ))
    def _proj():
        j = i - NP
        s = j // 2
        half = j % 2
        bs = qT_ref.shape[1]
        rh = w_ref.shape[0]
        rowr = lax.broadcasted_iota(jnp.int32, (bs, Bb), 0)
        colb = lax.broadcasted_iota(jnp.int32, (bs, Bb), 1)
        sel = (rowr == colb * S + s).astype(F32)
        ps = lax.dot_general(sel, acc_ref[...], (((1,), (0,)), ((), ())),
                             preferred_element_type=F32)
        qT_ref[pl.ds(half * rh, rh), :] += lax.dot_general(
            w_ref[...], ps, (((1,), (1,)), ((), ())),
            preferred_element_type=F32)

    @pl.when(i == NP + NJ - 1)
    def _fin_proj():
        q = jnp.transpose(qT_ref[...])
        preq_ref[...] = q.reshape(preq_ref.shape)

    @pl.when(i >= NP + NJ)
    def _dist():
        j = i - (NP + NJ)
        qT = qT_ref[...]

        @pl.when(j == 0)
        def _sqp():
            sqp_ref[...] = jnp.sum(qT * qT, axis=0, keepdims=True)

        c = c_ref[...]
        kb = c.shape[0]
        dotT = lax.dot_general(c, qT, (((1,), (0,)), ((), ())),
                               preferred_element_type=F32)
        cnorm = jnp.sum(c * c, axis=1, keepdims=True)
        logitsT = 2.0 * dotT - sqp_ref[...] - cnorm
        logits_ref[...] = jnp.transpose(logitsT).reshape(logits_ref.shape)

        rowid = lax.broadcasted_iota(jnp.int32, logitsT.shape, 0) + j * kb
        lmax = jnp.max(logitsT, axis=0, keepdims=True)
        larg = jnp.min(jnp.where(logitsT == lmax, rowid, jnp.int32(2**30)),
                       axis=0, keepdims=True)

        @pl.when(j == 0)
        def _first():
            bestv_ref[...] = lmax
            besti_ref[...] = larg

        @pl.when(j > 0)
        def _upd():
            take = lmax > bestv_ref[...]
            bestv_ref[...] = jnp.where(take, lmax, bestv_ref[...])
            besti_ref[...] = jnp.where(take, larg, besti_ref[...])

        @pl.when(j == NK - 1)
        def _fin():
            idx_ref[...] = besti_ref[...]


def _loss_body(q_ref, e_ref, cl_ref, bl_ref):
    d = e_ref[...] - q_ref[...]
    m = jnp.mean(d * d)
    cl_ref[...] = jnp.broadcast_to(m, (1, 1))
    bl_ref[...] = jnp.broadcast_to(m, (1, 1))


def _sc_gather_body(cb_hbm, idx_hbm, out_hbm, idx_v, rows_v, sem):
    wid = lax.axis_index("s") * 2 + lax.axis_index("c")

    @pl.when(wid < 8)
    def _():
        base = wid * 8
        pltpu.sync_copy(idx_hbm.at[pl.ds(base, 8)], idx_v)
        pltpu.async_copy(cb_hbm.at[idx_v], rows_v, sem).wait()
        pltpu.sync_copy(rows_v, out_hbm.at[pl.ds(base, 8)])


def kernel(hidden_states, attention_mask, W_slot, codebook):
    B, T, H = hidden_states.shape
    SH = W_slot.shape[0]
    S = SH // H
    K = codebook.shape[0]
    BS = B * S

    maskf = attention_mask.astype(F32)[:, :, None]

    Tb = 128
    Rb = 1024
    Kb = 1024
    NP = T // Tb
    NJ = SH // Rb
    NK = K // Kb

    body = functools.partial(_fused_body, NP=NP, NJ=NJ, NK=NK, Bb=B, S=S)
    pre_q, logits, idx2 = pl.pallas_call(
        body,
        grid=(NP + NJ + NK,),
        in_specs=[
            pl.BlockSpec((B, Tb, 1),
                         lambda i, NP=NP: (0, jnp.clip(i, 0, NP - 1), 0)),
            pl.BlockSpec((B, Tb, H),
                         lambda i, NP=NP: (0, jnp.clip(i, 0, NP - 1), 0)),
            pl.BlockSpec((Rb, H),
                         lambda i, NP=NP, NJ=NJ: (jnp.clip(i - NP, 0, NJ - 1), 0)),
            pl.BlockSpec((Kb, H),
                         lambda i, NP=NP, NJ=NJ, NK=NK:
                         (jnp.clip(i - NP - NJ, 0, NK - 1), 0)),
        ],
        out_specs=[
            pl.BlockSpec((B, S, H), lambda i: (0, 0, 0)),
            pl.BlockSpec((B, S, Kb),
                         lambda i, NP=NP, NJ=NJ, NK=NK:
                         (0, 0, jnp.clip(i - NP - NJ, 0, NK - 1))),
            pl.BlockSpec((1, BS), lambda i: (0, 0)),
        ],
        out_shape=[
            jax.ShapeDtypeStruct((B, S, H), F32),
            jax.ShapeDtypeStruct((B, S, K), F32),
            jax.ShapeDtypeStruct((1, BS), jnp.int32),
        ],
        scratch_shapes=[
            pltpu.VMEM((B, H), F32),
            pltpu.VMEM((B, 1), F32),
            pltpu.VMEM((H, BS), F32),
            pltpu.VMEM((1, BS), F32),
            pltpu.VMEM((1, BS), F32),
            pltpu.VMEM((1, BS), jnp.int32),
        ],
    )(maskf, hidden_states, W_slot, codebook)

    indices = idx2.reshape(B, S)
    q64 = pre_q.reshape(BS, H)

    mesh = plsc.VectorSubcoreMesh(core_axis_name="c", subcore_axis_name="s")
    embedded = pl.kernel(
        _sc_gather_body,
        mesh=mesh,
        out_type=jax.ShapeDtypeStruct((BS, H), F32),
        scratch_types=[
            pltpu.VMEM((8,), jnp.int32),
            pltpu.VMEM((8, H), F32),
            pltpu.SemaphoreType.DMA,
        ],
    )(codebook, idx2.reshape(BS))

    cl, bl = pl.pallas_call(
        _loss_body,
        out_shape=[
            jax.ShapeDtypeStruct((1, 1), F32),
            jax.ShapeDtypeStruct((1, 1), F32),
        ],
    )(q64, embedded)

    return (
        logits,
        indices,
        pre_q,
        embedded.reshape(B, S, H),
        cl.reshape(()),
        bl.reshape(()),
    )


# speedup vs baseline: 1.0162x; 1.0162x over previous
import functools

import jax
import jax.numpy as jnp
from jax import lax
from jax.experimental import pallas as pl
from jax.experimental.pallas import tpu as pltpu
from jax.experimental.pallas import tpu_sc as plsc

F32 = jnp.float32


def _fused_body(m_ref, x_ref, w_ref, c_ref, preq_ref, logits_ref, idx_ref,
                cl_ref, bl_ref,
                acc_ref, den_ref, qT_ref, sqp_ref, bestv_ref, besti_ref,
                *, NP, NJ, NK, Bb, S):
    i = pl.program_id(0)

    @pl.when(i == 0)
    def _init():
        acc_ref[...] = jnp.zeros_like(acc_ref)
        den_ref[...] = jnp.zeros_like(den_ref)
        qT_ref[...] = jnp.zeros_like(qT_ref)

    @pl.when(i < NP)
    def _pool():
        m = m_ref[...]
        x = x_ref[...]
        acc_ref[...] += jnp.sum(x * m, axis=1)
        den_ref[...] += jnp.sum(m[:, :, 0], axis=1, keepdims=True)

    @pl.when(i == NP - 1)
    def _fin_pool():
        acc_ref[...] = acc_ref[...] / jnp.clip(den_ref[...], 1.0, None)

    @pl.when((i >= NP) & (i < NP + NJ))
    def _proj():
        j = i - NP
        s = j // 2
        half = j % 2
        bs = qT_ref.shape[1]
        rh = w_ref.shape[0]
        rowr = lax.broadcasted_iota(jnp.int32, (bs, Bb), 0)
        colb = lax.broadcasted_iota(jnp.int32, (bs, Bb), 1)
        sel = (rowr == colb * S + s).astype(F32)
        ps = lax.dot_general(sel, acc_ref[...], (((1,), (0,)), ((), ())),
                             preferred_element_type=F32)
        qT_ref[pl.ds(half * rh, rh), :] += lax.dot_general(
            w_ref[...], ps, (((1,), (1,)), ((), ())),
            preferred_element_type=F32)

    @pl.when(i == NP + NJ - 1)
    def _fin_proj():
        q = jnp.transpose(qT_ref[...])
        preq_ref[...] = q.reshape(preq_ref.shape)

    @pl.when(i >= NP + NJ)
    def _dist():
        j = i - (NP + NJ)
        qT = qT_ref[...]

        @pl.when(j == 0)
        def _sqp():
            sqp_ref[...] = jnp.sum(qT * qT, axis=0, keepdims=True)

        c = c_ref[...]
        kb = c.shape[0]
        dotT = lax.dot_general(c, qT, (((1,), (0,)), ((), ())),
                               preferred_element_type=F32)
        cnorm = jnp.sum(c * c, axis=1, keepdims=True)
        logitsT = 2.0 * dotT - sqp_ref[...] - cnorm
        logits_ref[...] = jnp.transpose(logitsT).reshape(logits_ref.shape)

        rowid = lax.broadcasted_iota(jnp.int32, logitsT.shape, 0) + j * kb
        lmax = jnp.max(logitsT, axis=0, keepdims=True)
        larg = jnp.min(jnp.where(logitsT == lmax, rowid, jnp.int32(2**30)),
                       axis=0, keepdims=True)

        @pl.when(j == 0)
        def _first():
            bestv_ref[...] = lmax
            besti_ref[...] = larg

        @pl.when(j > 0)
        def _upd():
            take = lmax > bestv_ref[...]
            bestv_ref[...] = jnp.where(take, lmax, bestv_ref[...])
            besti_ref[...] = jnp.where(take, larg, besti_ref[...])

        @pl.when(j == NK - 1)
        def _fin():
            idx_ref[...] = besti_ref[...]
            n = bestv_ref.shape[1] * qT_ref.shape[0]
            m = -jnp.sum(bestv_ref[...]) * (1.0 / n)
            cl_ref[...] = jnp.broadcast_to(m, (1, 1))
            bl_ref[...] = jnp.broadcast_to(m, (1, 1))


def _sc_gather_body(cb_hbm, idx_hbm, out_hbm, idx_v, rows_v, sem):
    wid = lax.axis_index("s") * 2 + lax.axis_index("c")

    @pl.when(wid < 8)
    def _():
        base = wid * 8
        pltpu.sync_copy(idx_hbm.at[0, pl.ds(base, 8)], idx_v)
        pltpu.async_copy(cb_hbm.at[idx_v], rows_v, sem).wait()
        pltpu.sync_copy(rows_v, out_hbm.at[pl.ds(base, 8)])


def kernel(hidden_states, attention_mask, W_slot, codebook):
    B, T, H = hidden_states.shape
    SH = W_slot.shape[0]
    S = SH // H
    K = codebook.shape[0]
    BS = B * S

    maskf = attention_mask.astype(F32)[:, :, None]

    Tb = 128
    Rb = 1024
    Kb = 1024
    NP = T // Tb
    NJ = SH // Rb
    NK = K // Kb

    body = functools.partial(_fused_body, NP=NP, NJ=NJ, NK=NK, Bb=B, S=S)
    pre_q, logits, idx2, cl, bl = pl.pallas_call(
        body,
        grid=(NP + NJ + NK,),
        in_specs=[
            pl.BlockSpec((B, Tb, 1),
                         lambda i, NP=NP: (0, jnp.clip(i, 0, NP - 1), 0)),
            pl.BlockSpec((B, Tb, H),
                         lambda i, NP=NP: (0, jnp.clip(i, 0, NP - 1), 0)),
            pl.BlockSpec((Rb, H),
                         lambda i, NP=NP, NJ=NJ: (jnp.clip(i - NP, 0, NJ - 1), 0)),
            pl.BlockSpec((Kb, H),
                         lambda i, NP=NP, NJ=NJ, NK=NK:
                         (jnp.clip(i - NP - NJ, 0, NK - 1), 0)),
        ],
        out_specs=[
            pl.BlockSpec((B, S, H), lambda i: (0, 0, 0)),
            pl.BlockSpec((B, S, Kb),
                         lambda i, NP=NP, NJ=NJ, NK=NK:
                         (0, 0, jnp.clip(i - NP - NJ, 0, NK - 1))),
            pl.BlockSpec((1, BS), lambda i: (0, 0)),
            pl.BlockSpec((1, 1), lambda i: (0, 0)),
            pl.BlockSpec((1, 1), lambda i: (0, 0)),
        ],
        out_shape=[
            jax.ShapeDtypeStruct((B, S, H), F32),
            jax.ShapeDtypeStruct((B, S, K), F32),
            jax.ShapeDtypeStruct((1, BS), jnp.int32),
            jax.ShapeDtypeStruct((1, 1), F32),
            jax.ShapeDtypeStruct((1, 1), F32),
        ],
        scratch_shapes=[
            pltpu.VMEM((B, H), F32),
            pltpu.VMEM((B, 1), F32),
            pltpu.VMEM((H, BS), F32),
            pltpu.VMEM((1, BS), F32),
            pltpu.VMEM((1, BS), F32),
            pltpu.VMEM((1, BS), jnp.int32),
        ],
    )(maskf, hidden_states, W_slot, codebook)

    indices = idx2.reshape(B, S)

    mesh = plsc.VectorSubcoreMesh(core_axis_name="c", subcore_axis_name="s")
    embedded = pl.kernel(
        _sc_gather_body,
        mesh=mesh,
        out_type=jax.ShapeDtypeStruct((BS, H), F32),
        scratch_types=[
            pltpu.VMEM((8,), jnp.int32),
            pltpu.VMEM((8, H), F32),
            pltpu.SemaphoreType.DMA,
        ],
    )(codebook, idx2)

    return (
        logits,
        indices,
        pre_q,
        embedded.reshape(B, S, H),
        cl.reshape(()),
        bl.reshape(()),
    )
